# baseline (speedup 1.0000x reference)
import functools

import jax
import jax.numpy as jnp
from jax import lax
from jax.experimental import pallas as pl
from jax.experimental.pallas import tpu as pltpu
from jax.experimental.pallas import tpu_sc as plsc

N = 10000
N_PAD = 10240
E = 160000
D = 256
TAU0 = 0.5

NC = 2
NS = 16
NW = NC * NS
EPTA = 5120
EPTB = E - 31 * EPTA
SL = N_PAD // NS
MVB = 1024

_mesh = plsc.VectorSubcoreMesh(
    core_axis_name="c", subcore_axis_name="s", num_cores=NC, num_subcores=NS
)


def _fill(ref, n, value):
    vec = jnp.full((16,), value, dtype=ref.dtype)

    def body(i, carry):
        ref[pl.ds(i * 16, 16)] = vec
        return carry

    lax.fori_loop(0, n // 16, body, 0)


def _rowcopy(src2d, row, dst, n):
    def body(i, carry):
        dst[pl.ds(i * 16, 16)] = src2d[row, pl.ds(i * 16, 16)]
        return carry

    lax.fori_loop(0, n // 16, body, 0)


@functools.partial(
    pl.kernel,
    out_type=jax.ShapeDtypeStruct((NC, N_PAD), jnp.float32),
    mesh=_mesh,
    scratch_types=[
        pltpu.VMEM((2, EPTA), jnp.int32),
        pltpu.VMEM((2, EPTB), jnp.int32),
        pltpu.VMEM((EPTA,), jnp.int32),
        pltpu.VMEM((EPTB,), jnp.int32),
        pltpu.VMEM((EPTA,), jnp.float32),
        pltpu.VMEM((SL,), jnp.float32),
        pltpu.VMEM_SHARED((N_PAD,), jnp.float32),
    ],
)
def _deg_kernel(ei_hbm, out_hbm, ei_v, ei_b, dst1_v, dst1_b, ones_v, zeros_v, deg_sp):
    c = lax.axis_index("c")
    s = lax.axis_index("s")
    w = c * NS + s
    _fill(ones_v, EPTA, 1.0)
    _fill(zeros_v, SL, 0.0)
    pltpu.sync_copy(zeros_v, deg_sp.at[pl.ds(s * SL, SL)])

    @pl.when(w < NW - 1)
    def _():
        pltpu.sync_copy(ei_hbm.at[:, pl.ds(w * EPTA, EPTA)], ei_v)

    @pl.when(w == NW - 1)
    def _():
        pltpu.sync_copy(ei_hbm.at[:, pl.ds((NW - 1) * EPTA, EPTB)], ei_b)

    plsc.subcore_barrier()

    @pl.when(w < NW - 1)
    def _():
        _rowcopy(ei_v, 1, dst1_v, EPTA)
        pltpu.sync_copy(ones_v, deg_sp.at[dst1_v], add=True)

    @pl.when(w == NW - 1)
    def _():
        _rowcopy(ei_b, 1, dst1_b, EPTB)
        pltpu.sync_copy(ones_v.at[pl.ds(0, EPTB)], deg_sp.at[dst1_b],
                        add=True)

    plsc.subcore_barrier()
    pltpu.sync_copy(deg_sp.at[pl.ds(s * SL, SL)], out_hbm.at[c, pl.ds(s * SL, SL)])


def _rsqrt16(d):
    y = 1.0 / d
    hd = 0.5 * d
    for _ in range(20):
        y = y * (1.5 - hd * y * y)
    return y


@functools.partial(
    pl.kernel,
    out_type=[
        jax.ShapeDtypeStruct((NC, N_PAD), jnp.float32),
        jax.ShapeDtypeStruct((N_PAD,), jnp.float32),
        jax.ShapeDtypeStruct((N_PAD,), jnp.float32),
    ],
    mesh=_mesh,
    scratch_types=[
        pltpu.VMEM((2, EPTA), jnp.int32),
        pltpu.VMEM((2, EPTB), jnp.int32),
        pltpu.VMEM((EPTA,), jnp.int32),
        pltpu.VMEM((EPTA,), jnp.int32),
        pltpu.VMEM((EPTB,), jnp.int32),
        pltpu.VMEM((EPTB,), jnp.int32),
        pltpu.VMEM((EPTA,), jnp.float32),
        pltpu.VMEM((EPTB,), jnp.float32),
        pltpu.VMEM((SL,), jnp.float32),
        pltpu.VMEM((SL,), jnp.float32),
        pltpu.VMEM((SL,), jnp.float32),
        pltpu.VMEM((SL,), jnp.float32),
        pltpu.VMEM((SL,), jnp.float32),
        pltpu.VMEM((SL,), jnp.float32),
        pltpu.VMEM((SL,), jnp.float32),
        pltpu.VMEM_SHARED((N_PAD,), jnp.float32),
        pltpu.VMEM_SHARED((N_PAD,), jnp.float32),
    ],
)
def _edge_kernel(ei_hbm, deg2_hbm, h_hbm,
                 out_hbm, dinv_hbm, base_hbm,
                 ei_v, ei_b, src1_v, dst1_v, src1_b, dst1_b,
                 vals_v, vals_b, zeros_v, dega_v, degb_v,
                 h_v, p_v, dinv_v, base_v, p_sp, acc_sp):
    c = lax.axis_index("c")
    s = lax.axis_index("s")
    w = c * NS + s
    _fill(zeros_v, SL, 0.0)
    pltpu.sync_copy(zeros_v, acc_sp.at[pl.ds(s * SL, SL)])

    @pl.when(w < NW - 1)
    def _():
        pltpu.sync_copy(ei_hbm.at[:, pl.ds(w * EPTA, EPTA)], ei_v)

    @pl.when(w == NW - 1)
    def _():
        pltpu.sync_copy(ei_hbm.at[:, pl.ds((NW - 1) * EPTA, EPTB)], ei_b)

    pltpu.sync_copy(deg2_hbm.at[0, pl.ds(s * SL, SL)], dega_v)
    pltpu.sync_copy(deg2_hbm.at[1, pl.ds(s * SL, SL)], degb_v)
    pltpu.sync_copy(h_hbm.at[pl.ds(s * SL, SL)], h_v)

    def bodyv(i, carry):
        sl = pl.ds(i * 16, 16)
        d = dega_v[sl] + degb_v[sl] + 1.0
        y = _rsqrt16(d)
        hh = h_v[sl]
        p_v[sl] = y * hh
        dinv_v[sl] = y
        base_v[sl] = hh * (y * y)
        return carry

    lax.fori_loop(0, SL // 16, bodyv, 0)
    pltpu.sync_copy(p_v, p_sp.at[pl.ds(s * SL, SL)])

    @pl.when(c == 0)
    def _():
        pltpu.sync_copy(dinv_v, dinv_hbm.at[pl.ds(s * SL, SL)])
        pltpu.sync_copy(base_v, base_hbm.at[pl.ds(s * SL, SL)])

    plsc.subcore_barrier()

    @pl.when(w < NW - 1)
    def _():
        _rowcopy(ei_v, 0, src1_v, EPTA)
        _rowcopy(ei_v, 1, dst1_v, EPTA)
        pltpu.sync_copy(p_sp.at[src1_v], vals_v)
        pltpu.sync_copy(vals_v, acc_sp.at[dst1_v], add=True)

    @pl.when(w == NW - 1)
    def _():
        _rowcopy(ei_b, 0, src1_b, EPTB)
        _rowcopy(ei_b, 1, dst1_b, EPTB)
        pltpu.sync_copy(p_sp.at[src1_b], vals_b)
        pltpu.sync_copy(vals_b, acc_sp.at[dst1_b], add=True)

    plsc.subcore_barrier()
    pltpu.sync_copy(acc_sp.at[pl.ds(s * SL, SL)], out_hbm.at[c, pl.ds(s * SL, SL)])


def _mv_body(x_ref, w_ref, h_ref):
    h_ref[...] = jnp.dot(x_ref[...], w_ref[...],
                         preferred_element_type=jnp.float32)[:, 0]


def _fin_body(acc2_ref, dinv_ref, base_ref, t_ref):
    out = dinv_ref[...] * (acc2_ref[0, :] + acc2_ref[1, :]) + base_ref[...]
    sp = jax.nn.softplus(out) + TAU0
    t = 1.0 / sp
    t_ref[...] = jnp.where(jnp.isinf(t), 0.0, t)


def kernel(x, edge_index, edge_attr, W):
    h = pl.pallas_call(
        _mv_body,
        grid=(N_PAD // MVB,),
        in_specs=[
            pl.BlockSpec((MVB, D), lambda i: (i, 0)),
            pl.BlockSpec((D, 1), lambda i: (0, 0)),
        ],
        out_specs=pl.BlockSpec((MVB,), lambda i: (i,)),
        out_shape=jax.ShapeDtypeStruct((N_PAD,), jnp.float32),
    )(x, W)

    deg2 = _deg_kernel(edge_index)

    acc2, dinv, base = _edge_kernel(edge_index, deg2, h)

    temp = pl.pallas_call(
        _fin_body,
        out_shape=jax.ShapeDtypeStruct((N_PAD,), jnp.float32),
    )(acc2, dinv, base)

    return temp[:N, None]


# speedup vs baseline: 1.0377x; 1.0377x over previous
import functools

import jax
import jax.numpy as jnp
from jax import lax
from jax.experimental import pallas as pl
from jax.experimental.pallas import tpu as pltpu
from jax.experimental.pallas import tpu_sc as plsc

N = 10000
N_PAD = 10240
E = 160000
D = 256
TAU0 = 0.5

NC = 2
NS = 16
NW = NC * NS
EPTA = 5120
EPTB = E - 31 * EPTA
SL = N_PAD // NS
MVB = 1024

_mesh = plsc.VectorSubcoreMesh(
    core_axis_name="c", subcore_axis_name="s", num_cores=NC, num_subcores=NS
)


def _fill(ref, n, value):
    vec = jnp.full((16,), value, dtype=ref.dtype)

    def body(i, carry):
        ref[pl.ds(i * 16, 16)] = vec
        return carry

    lax.fori_loop(0, n // 16, body, 0)


def _rowcopy2(src2d, row, off, dst, n):
    def body(i, carry):
        dst[pl.ds(i * 16, 16)] = src2d[row, pl.ds(off + i * 16, 16)]
        return carry

    lax.fori_loop(0, n // 16, body, 0)


def _rowcopy(src2d, row, dst, n):
    def body(i, carry):
        dst[pl.ds(i * 16, 16)] = src2d[row, pl.ds(i * 16, 16)]
        return carry

    lax.fori_loop(0, n // 16, body, 0)


@functools.partial(
    pl.kernel,
    out_type=jax.ShapeDtypeStruct((NC, N_PAD), jnp.float32),
    mesh=_mesh,
    scratch_types=[
        pltpu.VMEM((2, EPTA), jnp.int32),
        pltpu.VMEM((2, EPTB), jnp.int32),
        pltpu.VMEM((EPTA,), jnp.int32),
        pltpu.VMEM((EPTB,), jnp.int32),
        pltpu.VMEM((EPTA,), jnp.float32),
        pltpu.VMEM((SL,), jnp.float32),
        pltpu.VMEM_SHARED((N_PAD,), jnp.float32),
        pltpu.SemaphoreType.DMA,
    ],
)
def _deg_kernel(ei_hbm, out_hbm, ei_v, ei_b, dst1_v, dst1_b, ones_v,
                zeros_v, deg_sp, sem):
    c = lax.axis_index("c")
    s = lax.axis_index("s")
    w = c * NS + s
    @pl.when(w < NW - 1)
    def _():
        cp = pltpu.async_copy(ei_hbm.at[:, pl.ds(w * EPTA, EPTA)], ei_v, sem)
        _fill(ones_v, EPTA, 1.0)
        _fill(zeros_v, SL, 0.0)
        cp.wait()

    @pl.when(w == NW - 1)
    def _():
        cp = pltpu.async_copy(ei_hbm.at[:, pl.ds((NW - 1) * EPTA, EPTB)],
                              ei_b, sem)
        _fill(ones_v, EPTA, 1.0)
        _fill(zeros_v, SL, 0.0)
        cp.wait()

    pltpu.sync_copy(zeros_v, deg_sp.at[pl.ds(s * SL, SL)])
    plsc.subcore_barrier()

    @pl.when(w < NW - 1)
    def _():
        _rowcopy(ei_v, 1, dst1_v, EPTA)
        pltpu.sync_copy(ones_v, deg_sp.at[dst1_v], add=True)

    @pl.when(w == NW - 1)
    def _():
        _rowcopy(ei_b, 1, dst1_b, EPTB)
        pltpu.sync_copy(ones_v.at[pl.ds(0, EPTB)], deg_sp.at[dst1_b],
                        add=True)

    plsc.subcore_barrier()
    pltpu.sync_copy(deg_sp.at[pl.ds(s * SL, SL)], out_hbm.at[c, pl.ds(s * SL, SL)])


def _rsqrt16(d):
    y = 1.0 / d
    hd = 0.5 * d
    for _ in range(20):
        y = y * (1.5 - hd * y * y)
    return y


@functools.partial(
    pl.kernel,
    out_type=[
        jax.ShapeDtypeStruct((NC, N_PAD), jnp.float32),
        jax.ShapeDtypeStruct((N_PAD,), jnp.float32),
        jax.ShapeDtypeStruct((N_PAD,), jnp.float32),
    ],
    mesh=_mesh,
    scratch_types=[
        pltpu.VMEM((2, EPTA), jnp.int32),
        pltpu.VMEM((2, EPTB), jnp.int32),
        pltpu.VMEM((EPTA // 2,), jnp.int32),
        pltpu.VMEM((EPTA // 2,), jnp.int32),
        pltpu.VMEM((EPTA // 2,), jnp.int32),
        pltpu.VMEM((EPTA // 2,), jnp.int32),
        pltpu.VMEM((EPTB,), jnp.int32),
        pltpu.VMEM((EPTB,), jnp.int32),
        pltpu.VMEM((EPTA // 2,), jnp.float32),
        pltpu.VMEM((EPTA // 2,), jnp.float32),
        pltpu.VMEM((EPTB,), jnp.float32),
        pltpu.VMEM((SL,), jnp.float32),
        pltpu.VMEM((SL,), jnp.float32),
        pltpu.VMEM((SL,), jnp.float32),
        pltpu.VMEM((SL,), jnp.float32),
        pltpu.VMEM((SL,), jnp.float32),
        pltpu.VMEM((SL,), jnp.float32),
        pltpu.VMEM((SL,), jnp.float32),
        pltpu.VMEM_SHARED((N_PAD,), jnp.float32),
        pltpu.VMEM_SHARED((N_PAD,), jnp.float32),
        pltpu.SemaphoreType.DMA,
        pltpu.SemaphoreType.DMA,
        pltpu.SemaphoreType.DMA,
    ],
)
def _edge_kernel(ei_hbm, deg2_hbm, h_hbm,
                 out_hbm, dinv_hbm, base_hbm,
                 ei_v, ei_b, src_h0, src_h1, dst_h0, dst_h1, src1_b, dst1_b,
                 vals0, vals1, vals_b, zeros_v, dega_v, degb_v,
                 h_v, p_v, dinv_v, base_v, p_sp, acc_sp, sem, semg, sems):
    c = lax.axis_index("c")
    s = lax.axis_index("s")
    w = c * NS + s
    da = pltpu.async_copy(deg2_hbm.at[0, pl.ds(s * SL, SL)], dega_v, sem)
    db = pltpu.async_copy(deg2_hbm.at[1, pl.ds(s * SL, SL)], degb_v, sem)
    dh = pltpu.async_copy(h_hbm.at[pl.ds(s * SL, SL)], h_v, sem)

    @pl.when(w < NW - 1)
    def _():
        pltpu.sync_copy(ei_hbm.at[:, pl.ds(w * EPTA, EPTA)], ei_v)

    @pl.when(w == NW - 1)
    def _():
        pltpu.sync_copy(ei_hbm.at[:, pl.ds((NW - 1) * EPTA, EPTB)], ei_b)

    _fill(zeros_v, SL, 0.0)
    pltpu.sync_copy(zeros_v, acc_sp.at[pl.ds(s * SL, SL)])
    da.wait()
    db.wait()
    dh.wait()

    def bodyv(i, carry):
        sl = pl.ds(i * 16, 16)
        d = dega_v[sl] + degb_v[sl] + 1.0
        y = _rsqrt16(d)
        hh = h_v[sl]
        p_v[sl] = y * hh
        dinv_v[sl] = y
        base_v[sl] = hh * (y * y)
        return carry

    lax.fori_loop(0, SL // 16, bodyv, 0)
    pltpu.sync_copy(p_v, p_sp.at[pl.ds(s * SL, SL)])

    @pl.when(c == 0)
    def _():
        pltpu.sync_copy(dinv_v, dinv_hbm.at[pl.ds(s * SL, SL)])
        pltpu.sync_copy(base_v, base_hbm.at[pl.ds(s * SL, SL)])

    plsc.subcore_barrier()

    @pl.when(w < NW - 1)
    def _():
        H = EPTA // 2
        _rowcopy2(ei_v, 0, 0, src_h0, H)
        _rowcopy2(ei_v, 0, H, src_h1, H)
        _rowcopy2(ei_v, 1, 0, dst_h0, H)
        _rowcopy2(ei_v, 1, H, dst_h1, H)
        g0 = pltpu.async_copy(p_sp.at[src_h0], vals0, semg)
        g0.wait()
        s0 = pltpu.async_copy(vals0, acc_sp.at[dst_h0], sems, add=True)
        g1 = pltpu.async_copy(p_sp.at[src_h1], vals1, semg)
        g1.wait()
        s1 = pltpu.async_copy(vals1, acc_sp.at[dst_h1], sems, add=True)
        s0.wait()
        s1.wait()

    @pl.when(w == NW - 1)
    def _():
        _rowcopy(ei_b, 0, src1_b, EPTB)
        _rowcopy(ei_b, 1, dst1_b, EPTB)
        pltpu.sync_copy(p_sp.at[src1_b], vals_b)
        pltpu.sync_copy(vals_b, acc_sp.at[dst1_b], add=True)

    plsc.subcore_barrier()
    pltpu.sync_copy(acc_sp.at[pl.ds(s * SL, SL)], out_hbm.at[c, pl.ds(s * SL, SL)])


def _mv_body(x_ref, w_ref, h_ref):
    h_ref[...] = jnp.dot(x_ref[...], w_ref[...],
                         preferred_element_type=jnp.float32)[:, 0]


def _fin_body(acc2_ref, dinv_ref, base_ref, t_ref):
    out = dinv_ref[...] * (acc2_ref[0, :] + acc2_ref[1, :]) + base_ref[...]
    sp = jax.nn.softplus(out) + TAU0
    t = 1.0 / sp
    t_ref[...] = jnp.where(jnp.isinf(t), 0.0, t)


def kernel(x, edge_index, edge_attr, W):
    h = pl.pallas_call(
        _mv_body,
        grid=(N_PAD // MVB,),
        in_specs=[
            pl.BlockSpec((MVB, D), lambda i: (i, 0)),
            pl.BlockSpec((D, 1), lambda i: (0, 0)),
        ],
        out_specs=pl.BlockSpec((MVB,), lambda i: (i,)),
        out_shape=jax.ShapeDtypeStruct((N_PAD,), jnp.float32),
    )(x, W)

    deg2 = _deg_kernel(edge_index)

    acc2, dinv, base = _edge_kernel(edge_index, deg2, h)

    temp = pl.pallas_call(
        _fin_body,
        out_shape=jax.ShapeDtypeStruct((N_PAD,), jnp.float32),
    )(acc2, dinv, base)

    return temp[:N, None]


# speedup vs baseline: 1.0580x; 1.0196x over previous
import functools

import jax
import jax.numpy as jnp
from jax import lax
from jax.experimental import pallas as pl
from jax.experimental.pallas import tpu as pltpu
from jax.experimental.pallas import tpu_sc as plsc

N = 10000
N_PAD = 10240
E = 160000
D = 256
TAU0 = 0.5

NC = 2
NS = 16
NW = NC * NS
EPTA = 5120
EPTB = E - 31 * EPTA
SL = N_PAD // NS
MVB = 1024

_mesh = plsc.VectorSubcoreMesh(
    core_axis_name="c", subcore_axis_name="s", num_cores=NC, num_subcores=NS
)


def _fill(ref, n, value):
    vec = jnp.full((16,), value, dtype=ref.dtype)

    def body(i, carry):
        ref[pl.ds(i * 16, 16)] = vec
        return carry

    lax.fori_loop(0, n // 16, body, 0)


def _rowcopy2(src2d, row, off, dst, n):
    def body(i, carry):
        dst[pl.ds(i * 16, 16)] = src2d[row, pl.ds(off + i * 16, 16)]
        return carry

    lax.fori_loop(0, n // 16, body, 0)


def _rowcopy(src2d, row, dst, n):
    def body(i, carry):
        dst[pl.ds(i * 16, 16)] = src2d[row, pl.ds(i * 16, 16)]
        return carry

    lax.fori_loop(0, n // 16, body, 0)


@functools.partial(
    pl.kernel,
    out_type=jax.ShapeDtypeStruct((NC, N_PAD), jnp.float32),
    mesh=_mesh,
    scratch_types=[
        pltpu.VMEM((2, EPTA), jnp.int32),
        pltpu.VMEM((2, EPTB), jnp.int32),
        pltpu.VMEM((EPTA,), jnp.int32),
        pltpu.VMEM((EPTB,), jnp.int32),
        pltpu.VMEM((EPTA,), jnp.float32),
        pltpu.VMEM((SL,), jnp.float32),
        pltpu.VMEM_SHARED((N_PAD,), jnp.float32),
        pltpu.SemaphoreType.DMA,
    ],
)
def _deg_kernel(ei_hbm, out_hbm, ei_v, ei_b, dst1_v, dst1_b, ones_v,
                zeros_v, deg_sp, sem):
    c = lax.axis_index("c")
    s = lax.axis_index("s")
    w = c * NS + s
    @pl.when(w < NW - 1)
    def _():
        cp = pltpu.async_copy(ei_hbm.at[:, pl.ds(w * EPTA, EPTA)], ei_v, sem)
        _fill(ones_v, EPTA, 1.0)
        _fill(zeros_v, SL, 0.0)
        cp.wait()

    @pl.when(w == NW - 1)
    def _():
        cp = pltpu.async_copy(ei_hbm.at[:, pl.ds((NW - 1) * EPTA, EPTB)],
                              ei_b, sem)
        _fill(ones_v, EPTA, 1.0)
        _fill(zeros_v, SL, 0.0)
        cp.wait()

    pltpu.sync_copy(zeros_v, deg_sp.at[pl.ds(s * SL, SL)])
    plsc.subcore_barrier()

    @pl.when(w < NW - 1)
    def _():
        _rowcopy(ei_v, 1, dst1_v, EPTA)
        pltpu.sync_copy(ones_v, deg_sp.at[dst1_v], add=True)

    @pl.when(w == NW - 1)
    def _():
        _rowcopy(ei_b, 1, dst1_b, EPTB)
        pltpu.sync_copy(ones_v.at[pl.ds(0, EPTB)], deg_sp.at[dst1_b],
                        add=True)

    plsc.subcore_barrier()
    pltpu.sync_copy(deg_sp.at[pl.ds(s * SL, SL)], out_hbm.at[c, pl.ds(s * SL, SL)])


def _rsqrt16(d):
    y = 1.0 / d
    hd = 0.5 * d
    for _ in range(20):
        y = y * (1.5 - hd * y * y)
    return y


@functools.partial(
    pl.kernel,
    out_type=[
        jax.ShapeDtypeStruct((NC, N_PAD), jnp.float32),
        jax.ShapeDtypeStruct((N_PAD,), jnp.float32),
        jax.ShapeDtypeStruct((N_PAD,), jnp.float32),
    ],
    mesh=_mesh,
    scratch_types=[
        pltpu.VMEM((2, EPTA), jnp.int32),
        pltpu.VMEM((2, EPTB), jnp.int32),
        pltpu.VMEM((EPTA // 2,), jnp.int32),
        pltpu.VMEM((EPTA // 2,), jnp.int32),
        pltpu.VMEM((EPTA // 2,), jnp.int32),
        pltpu.VMEM((EPTA // 2,), jnp.int32),
        pltpu.VMEM((EPTB,), jnp.int32),
        pltpu.VMEM((EPTB,), jnp.int32),
        pltpu.VMEM((EPTA // 2,), jnp.float32),
        pltpu.VMEM((EPTA // 2,), jnp.float32),
        pltpu.VMEM((EPTB,), jnp.float32),
        pltpu.VMEM((SL,), jnp.float32),
        pltpu.VMEM((SL,), jnp.float32),
        pltpu.VMEM((SL,), jnp.float32),
        pltpu.VMEM((SL,), jnp.float32),
        pltpu.VMEM((SL,), jnp.float32),
        pltpu.VMEM((SL,), jnp.float32),
        pltpu.VMEM((SL,), jnp.float32),
        pltpu.VMEM_SHARED((N_PAD,), jnp.float32),
        pltpu.VMEM_SHARED((N_PAD,), jnp.float32),
        pltpu.SemaphoreType.DMA,
        pltpu.SemaphoreType.DMA,
        pltpu.SemaphoreType.DMA,
    ],
)
def _edge_kernel(ei_hbm, deg2_hbm, h_hbm,
                 out_hbm, dinv_hbm, base_hbm,
                 ei_v, ei_b, src_h0, src_h1, dst_h0, dst_h1, src1_b, dst1_b,
                 vals0, vals1, vals_b, zeros_v, dega_v, degb_v,
                 h_v, p_v, dinv_v, base_v, p_sp, acc_sp, sem, semg, sems):
    c = lax.axis_index("c")
    s = lax.axis_index("s")
    w = c * NS + s
    da = pltpu.async_copy(deg2_hbm.at[0, pl.ds(s * SL, SL)], dega_v, sem)
    db = pltpu.async_copy(deg2_hbm.at[1, pl.ds(s * SL, SL)], degb_v, sem)
    dh = pltpu.async_copy(h_hbm.at[pl.ds(s * SL, SL)], h_v, sem)

    @pl.when(w < NW - 1)
    def _():
        pltpu.sync_copy(ei_hbm.at[:, pl.ds(w * EPTA, EPTA)], ei_v)
        _rowcopy2(ei_v, 0, 0, src_h0, EPTA // 2)

    @pl.when(w == NW - 1)
    def _():
        pltpu.sync_copy(ei_hbm.at[:, pl.ds((NW - 1) * EPTA, EPTB)], ei_b)

    _fill(zeros_v, SL, 0.0)
    pltpu.sync_copy(zeros_v, acc_sp.at[pl.ds(s * SL, SL)])
    da.wait()
    db.wait()
    dh.wait()

    def bodyv(i, carry):
        sl = pl.ds(i * 16, 16)
        d = dega_v[sl] + degb_v[sl] + 1.0
        y = _rsqrt16(d)
        hh = h_v[sl]
        p_v[sl] = y * hh
        dinv_v[sl] = y
        base_v[sl] = hh * (y * y)
        return carry

    lax.fori_loop(0, SL // 16, bodyv, 0)
    pltpu.sync_copy(p_v, p_sp.at[pl.ds(s * SL, SL)])

    @pl.when(c == 0)
    def _():
        pltpu.sync_copy(dinv_v, dinv_hbm.at[pl.ds(s * SL, SL)])
        pltpu.sync_copy(base_v, base_hbm.at[pl.ds(s * SL, SL)])

    plsc.subcore_barrier()

    @pl.when(w < NW - 1)
    def _():
        H = EPTA // 2
        g0 = pltpu.async_copy(p_sp.at[src_h0], vals0, semg)
        _rowcopy2(ei_v, 0, H, src_h1, H)
        _rowcopy2(ei_v, 1, 0, dst_h0, H)
        _rowcopy2(ei_v, 1, H, dst_h1, H)
        g0.wait()
        s0 = pltpu.async_copy(vals0, acc_sp.at[dst_h0], sems, add=True)
        g1 = pltpu.async_copy(p_sp.at[src_h1], vals1, semg)
        g1.wait()
        s1 = pltpu.async_copy(vals1, acc_sp.at[dst_h1], sems, add=True)
        s0.wait()
        s1.wait()

    @pl.when(w == NW - 1)
    def _():
        _rowcopy(ei_b, 0, src1_b, EPTB)
        _rowcopy(ei_b, 1, dst1_b, EPTB)
        pltpu.sync_copy(p_sp.at[src1_b], vals_b)
        pltpu.sync_copy(vals_b, acc_sp.at[dst1_b], add=True)

    plsc.subcore_barrier()
    pltpu.sync_copy(acc_sp.at[pl.ds(s * SL, SL)], out_hbm.at[c, pl.ds(s * SL, SL)])


def _mv_body(x_ref, w_ref, h_ref):
    h_ref[...] = jnp.dot(x_ref[...], w_ref[...],
                         preferred_element_type=jnp.float32)[:, 0]


def _fin_body(acc2_ref, dinv_ref, base_ref, t_ref):
    out = dinv_ref[...] * (acc2_ref[0, :] + acc2_ref[1, :]) + base_ref[...]
    sp = jax.nn.softplus(out) + TAU0
    t = 1.0 / sp
    t_ref[...] = jnp.where(jnp.isinf(t), 0.0, t)


def kernel(x, edge_index, edge_attr, W):
    h = pl.pallas_call(
        _mv_body,
        grid=(N_PAD // MVB,),
        in_specs=[
            pl.BlockSpec((MVB, D), lambda i: (i, 0)),
            pl.BlockSpec((D, 1), lambda i: (0, 0)),
        ],
        out_specs=pl.BlockSpec((MVB,), lambda i: (i,)),
        out_shape=jax.ShapeDtypeStruct((N_PAD,), jnp.float32),
    )(x, W)

    deg2 = _deg_kernel(edge_index)

    acc2, dinv, base = _edge_kernel(edge_index, deg2, h)

    temp = pl.pallas_call(
        _fin_body,
        out_shape=jax.ShapeDtypeStruct((N_PAD,), jnp.float32),
    )(acc2, dinv, base)

    return temp[:N, None]
